# baseline (speedup 1.0000x reference)
import functools

import jax
import jax.numpy as jnp
from jax import lax
from jax.experimental import pallas as pl
from jax.experimental.pallas import tpu as pltpu
from jax.experimental.pallas import tpu_sc as plsc

NUM_CORES = 2
NUM_SUBCORES = 16
NUM_WORKERS = NUM_CORES * NUM_SUBCORES

CHUNK = 320
NBUF = 3


def _body(idx_hbm, table_hbm, out_hbm, *refs, b_per_w):
    idx_v = refs[0]
    bufs = refs[1:1 + NBUF]
    gsems = refs[1 + NBUF:1 + 2 * NBUF]
    ssems = refs[1 + 2 * NBUF:1 + 3 * NBUF]

    wid = lax.axis_index("s") * NUM_CORES + lax.axis_index("c")
    base = wid * b_per_w
    pltpu.sync_copy(idx_hbm.at[pl.ds(base, b_per_w)], idx_v)

    n = b_per_w // CHUNK

    def gather(j):
        b = j % NBUF
        return pltpu.make_async_copy(
            table_hbm.at[idx_v.at[pl.ds(j * CHUNK, CHUNK)]], bufs[b], gsems[b])

    def store(j):
        b = j % NBUF
        return pltpu.make_async_copy(
            bufs[b], out_hbm.at[pl.ds(base + j * CHUNK, CHUNK)], ssems[b])

    gather(0).start()
    for j in range(n):
        if j + 1 < n:
            if j + 1 - NBUF >= 0:
                store(j + 1 - NBUF).wait()
            gather(j + 1).start()
        gather(j).wait()
        store(j).start()
    for j in range(max(0, n - NBUF), n):
        store(j).wait()


def kernel(x, weight):
    n0, n1 = x.shape
    d = weight.shape[1]
    flat = x.T.reshape(-1).astype(jnp.int32)
    b = flat.shape[0]
    assert b % NUM_WORKERS == 0
    b_per_w = b // NUM_WORKERS
    assert b_per_w % CHUNK == 0

    mesh = plsc.VectorSubcoreMesh(core_axis_name="c", subcore_axis_name="s")
    scratch = (
        [pltpu.VMEM((b_per_w,), jnp.int32)]
        + [pltpu.VMEM((CHUNK, d), jnp.float32) for _ in range(NBUF)]
        + [pltpu.SemaphoreType.DMA for _ in range(2 * NBUF)]
    )
    k = pl.kernel(
        functools.partial(_body, b_per_w=b_per_w),
        out_type=jax.ShapeDtypeStruct((b, d), jnp.float32),
        mesh=mesh,
        scratch_types=scratch,
    )
    out = k(flat, weight)
    return out.reshape(n1, n0, d).transpose(1, 0, 2)


# speedup vs baseline: 1.0209x; 1.0209x over previous
import functools

import jax
import jax.numpy as jnp
from jax import lax
from jax.experimental import pallas as pl
from jax.experimental.pallas import tpu as pltpu
from jax.experimental.pallas import tpu_sc as plsc

NUM_CORES = 2
NUM_SUBCORES = 16
NUM_WORKERS = NUM_CORES * NUM_SUBCORES

CHUNK = 400
NBUF = 2


def _body(idx_hbm, table_hbm, out_hbm, *refs, b_per_w):
    idx_v = refs[0]
    bufs = refs[1:1 + NBUF]
    gsems = refs[1 + NBUF:1 + 2 * NBUF]
    ssems = refs[1 + 2 * NBUF:1 + 3 * NBUF]

    wid = lax.axis_index("s") * NUM_CORES + lax.axis_index("c")
    base = wid * b_per_w
    pltpu.sync_copy(idx_hbm.at[pl.ds(base, b_per_w)], idx_v)

    n = b_per_w // CHUNK

    def gather(j):
        b = j % NBUF
        return pltpu.make_async_copy(
            table_hbm.at[idx_v.at[pl.ds(j * CHUNK, CHUNK)]], bufs[b], gsems[b])

    def store(j):
        b = j % NBUF
        return pltpu.make_async_copy(
            bufs[b], out_hbm.at[pl.ds(base + j * CHUNK, CHUNK)], ssems[b])

    gather(0).start()
    for j in range(n):
        if j + 1 < n:
            if j + 1 - NBUF >= 0:
                store(j + 1 - NBUF).wait()
            gather(j + 1).start()
        gather(j).wait()
        store(j).start()
    for j in range(max(0, n - NBUF), n):
        store(j).wait()


def kernel(x, weight):
    n0, n1 = x.shape
    d = weight.shape[1]
    flat = x.T.reshape(-1).astype(jnp.int32)
    b = flat.shape[0]
    assert b % NUM_WORKERS == 0
    b_per_w = b // NUM_WORKERS
    assert b_per_w % CHUNK == 0

    mesh = plsc.VectorSubcoreMesh(core_axis_name="c", subcore_axis_name="s")
    scratch = (
        [pltpu.VMEM((b_per_w,), jnp.int32)]
        + [pltpu.VMEM((CHUNK, d), jnp.float32) for _ in range(NBUF)]
        + [pltpu.SemaphoreType.DMA for _ in range(2 * NBUF)]
    )
    k = pl.kernel(
        functools.partial(_body, b_per_w=b_per_w),
        out_type=jax.ShapeDtypeStruct((b, d), jnp.float32),
        mesh=mesh,
        scratch_types=scratch,
    )
    out = k(flat, weight)
    return out.reshape(n1, n0, d).transpose(1, 0, 2)
